# baseline (speedup 1.0000x reference)
import numpy as np
import jax
import jax.numpy as jnp
from jax.experimental import pallas as pl

NE = 8192
D = 64
NQ = 4
ROWS = 4 * 24 * 24
R = 64
TILES = ROWS // R

_ROT_A = (13, 15, 26, 6)
_ROT_B = (17, 29, 16, 24)


def _np_threefry2x32(k1, k2, x0, x1):
    k1 = np.uint32(k1); k2 = np.uint32(k2)
    ks = (k1, k2, np.uint32(k1 ^ k2 ^ np.uint32(0x1BD11BDA)))
    x0 = np.uint32(x0); x1 = np.uint32(x1)

    def rl(x, r):
        return np.uint32((x << np.uint32(r)) | (x >> np.uint32(32 - r)))

    def rounds(x0, x1, rs):
        for r in rs:
            x0 = np.uint32(x0 + x1)
            x1 = np.uint32(x0 ^ rl(x1, r))
        return x0, x1

    x0 = np.uint32(x0 + ks[0]); x1 = np.uint32(x1 + ks[1])
    x0, x1 = rounds(x0, x1, _ROT_A)
    x0 = np.uint32(x0 + ks[1]); x1 = np.uint32(x1 + ks[2] + np.uint32(1))
    x0, x1 = rounds(x0, x1, _ROT_B)
    x0 = np.uint32(x0 + ks[2]); x1 = np.uint32(x1 + ks[0] + np.uint32(2))
    x0, x1 = rounds(x0, x1, _ROT_A)
    x0 = np.uint32(x0 + ks[0]); x1 = np.uint32(x1 + ks[1] + np.uint32(3))
    x0, x1 = rounds(x0, x1, _ROT_B)
    x0 = np.uint32(x0 + ks[1]); x1 = np.uint32(x1 + ks[2] + np.uint32(4))
    x0, x1 = rounds(x0, x1, _ROT_A)
    x0 = np.uint32(x0 + ks[2]); x1 = np.uint32(x1 + ks[0] + np.uint32(5))
    return x0, x1


def _step_keys():
    np.seterr(over="ignore")
    base = (np.uint32(0), np.uint32(1234))
    keys = []
    for i in range(NQ):
        o0, o1 = _np_threefry2x32(base[0], base[1], np.uint32(0), np.uint32(i))
        keys.append((int(o0), int(o1)))
    return keys


_KEYS = _step_keys()
_TINY = np.float32(np.finfo(np.float32).tiny)


def _jnp_threefry_bits(k1, k2, x1):
    ks = (np.uint32(k1), np.uint32(k2),
          np.uint32(np.uint32(k1) ^ np.uint32(k2) ^ np.uint32(0x1BD11BDA)))

    def rl(x, r):
        return (x << np.uint32(r)) | (x >> np.uint32(32 - r))

    def rounds(x0, x1, rs):
        for r in rs:
            x0 = x0 + x1
            x1 = x0 ^ rl(x1, r)
        return x0, x1

    x0 = jnp.full_like(x1, ks[0])
    x1 = x1 + ks[1]
    x0, x1 = rounds(x0, x1, _ROT_A)
    x0 = x0 + ks[1]; x1 = x1 + np.uint32(ks[2] + np.uint32(1))
    x0, x1 = rounds(x0, x1, _ROT_B)
    x0 = x0 + ks[2]; x1 = x1 + np.uint32(ks[0] + np.uint32(2))
    x0, x1 = rounds(x0, x1, _ROT_A)
    x0 = x0 + ks[0]; x1 = x1 + np.uint32(ks[1] + np.uint32(3))
    x0, x1 = rounds(x0, x1, _ROT_B)
    x0 = x0 + ks[1]; x1 = x1 + np.uint32(ks[2] + np.uint32(4))
    x0, x1 = rounds(x0, x1, _ROT_A)
    x0 = x0 + ks[2]; x1 = x1 + np.uint32(ks[0] + np.uint32(5))
    return x0 ^ x1


def _rvq_kernel(x_ref, cw_ref, cn_ref, quant_ref, probs_ref, idx_ref, loss_ref):
    t = pl.program_id(0)
    fr = x_ref[...]
    cw = cw_ref[...]
    cn = cn_ref[...]

    ci = jax.lax.broadcasted_iota(jnp.int32, (R, NE), 1)
    li = jax.lax.broadcasted_iota(jnp.int32, (R, NE), 0)
    base = t * (R * NE)
    p_u32 = (base + li * NE + ci).astype(jnp.uint32)

    quant = jnp.zeros((R, D), jnp.float32)
    loss_acc = jnp.float32(0.0)

    for s in range(NQ):
        bits = _jnp_threefry_bits(_KEYS[s][0], _KEYS[s][1], p_u32)
        fb = (bits >> np.uint32(9)) | np.uint32(0x3F800000)
        f = jax.lax.bitcast_convert_type(fb, jnp.float32) - jnp.float32(1.0)
        u = jnp.maximum(_TINY, f * (jnp.float32(1.0) - _TINY) + _TINY)
        g = -jnp.log(-jnp.log(u))

        rn = jnp.sum(fr * fr, axis=1, keepdims=True)
        m = jax.lax.dot_general(fr, cw, (((1,), (1,)), ((), ())))
        d = (rn + cn) - 2.0 * m
        logits = -d

        lmax = jnp.max(logits, axis=1, keepdims=True)
        e = jnp.exp(logits - lmax)
        probs = e / jnp.sum(e, axis=1, keepdims=True)
        probs_ref[s, :, :] = probs

        v = g + logits
        vmax = jnp.max(v, axis=1, keepdims=True)
        idx = jnp.min(jnp.where(v == vmax, ci, NE), axis=1, keepdims=True)
        idx_ref[s] = idx

        onehot = (ci == idx).astype(jnp.float32)
        qr = jax.lax.dot_general(onehot, cw, (((1,), (0,)), ((), ())))

        quant = quant + qr
        diff = qr - fr
        loss_acc = loss_acc + jnp.sum(diff * diff)
        fr = fr - qr

    quant_ref[...] = quant

    @pl.when(t == 0)
    def _():
        loss_ref[...] = jnp.zeros((1, 1), jnp.float32)

    loss_ref[...] += loss_acc.reshape(1, 1)


def kernel(x, codebook_weight):
    xt = jnp.transpose(x, (0, 2, 3, 1)).reshape(ROWS, D)
    cn = jnp.sum(codebook_weight ** 2, axis=1).reshape(1, NE)

    quant, probs, idx, loss = pl.pallas_call(
        _rvq_kernel,
        grid=(TILES,),
        in_specs=[
            pl.BlockSpec((R, D), lambda t: (t, 0)),
            pl.BlockSpec((NE, D), lambda t: (0, 0)),
            pl.BlockSpec((1, NE), lambda t: (0, 0)),
        ],
        out_specs=[
            pl.BlockSpec((R, D), lambda t: (t, 0)),
            pl.BlockSpec((NQ, R, NE), lambda t: (0, t, 0)),
            pl.BlockSpec((NQ, R, 1), lambda t: (0, t, 0)),
            pl.BlockSpec((1, 1), lambda t: (0, 0)),
        ],
        out_shape=[
            jax.ShapeDtypeStruct((ROWS, D), jnp.float32),
            jax.ShapeDtypeStruct((NQ, ROWS, NE), jnp.float32),
            jax.ShapeDtypeStruct((NQ, ROWS, 1), jnp.int32),
            jax.ShapeDtypeStruct((1, 1), jnp.float32),
        ],
    )(xt, codebook_weight, cn)

    B, C, H, W = x.shape
    quantized = jnp.transpose(quant.reshape(B, H, W, C), (0, 3, 1, 2))
    total_loss = (jnp.float32(1.25) * loss[0, 0] / jnp.float32(x.size))
    indices = tuple(idx[s, :, 0].reshape(B, H, W) for s in range(NQ))
    soft_targets = tuple(probs[s].reshape(B, H, W, NE) for s in range(NQ))
    return (quantized, total_loss, indices, soft_targets)


# speedup vs baseline: 1.0009x; 1.0009x over previous
import numpy as np
import jax
import jax.numpy as jnp
from jax.experimental import pallas as pl
from jax.experimental.pallas import tpu as pltpu

NE = 8192
D = 64
NQ = 4
ROWS = 4 * 24 * 24
R = 64
TILES = ROWS // R

_ROT_A = (13, 15, 26, 6)
_ROT_B = (17, 29, 16, 24)


def _np_threefry2x32(k1, k2, x0, x1):
    k1 = np.uint32(k1); k2 = np.uint32(k2)
    ks = (k1, k2, np.uint32(k1 ^ k2 ^ np.uint32(0x1BD11BDA)))
    x0 = np.uint32(x0); x1 = np.uint32(x1)

    def rl(x, r):
        return np.uint32((x << np.uint32(r)) | (x >> np.uint32(32 - r)))

    def rounds(x0, x1, rs):
        for r in rs:
            x0 = np.uint32(x0 + x1)
            x1 = np.uint32(x0 ^ rl(x1, r))
        return x0, x1

    x0 = np.uint32(x0 + ks[0]); x1 = np.uint32(x1 + ks[1])
    x0, x1 = rounds(x0, x1, _ROT_A)
    x0 = np.uint32(x0 + ks[1]); x1 = np.uint32(x1 + ks[2] + np.uint32(1))
    x0, x1 = rounds(x0, x1, _ROT_B)
    x0 = np.uint32(x0 + ks[2]); x1 = np.uint32(x1 + ks[0] + np.uint32(2))
    x0, x1 = rounds(x0, x1, _ROT_A)
    x0 = np.uint32(x0 + ks[0]); x1 = np.uint32(x1 + ks[1] + np.uint32(3))
    x0, x1 = rounds(x0, x1, _ROT_B)
    x0 = np.uint32(x0 + ks[1]); x1 = np.uint32(x1 + ks[2] + np.uint32(4))
    x0, x1 = rounds(x0, x1, _ROT_A)
    x0 = np.uint32(x0 + ks[2]); x1 = np.uint32(x1 + ks[0] + np.uint32(5))
    return x0, x1


def _step_keys():
    np.seterr(over="ignore")
    base = (np.uint32(0), np.uint32(1234))
    keys = []
    for i in range(NQ):
        o0, o1 = _np_threefry2x32(base[0], base[1], np.uint32(0), np.uint32(i))
        keys.append((int(o0), int(o1)))
    return keys


_KEYS = _step_keys()
_TINY = np.float32(np.finfo(np.float32).tiny)


def _jnp_threefry_bits(k1, k2, x1):
    ks = (np.uint32(k1), np.uint32(k2),
          np.uint32(np.uint32(k1) ^ np.uint32(k2) ^ np.uint32(0x1BD11BDA)))

    def rl(x, r):
        return (x << np.uint32(r)) | (x >> np.uint32(32 - r))

    def rounds(x0, x1, rs):
        for r in rs:
            x0 = x0 + x1
            x1 = x0 ^ rl(x1, r)
        return x0, x1

    x0 = jnp.full_like(x1, ks[0])
    x1 = x1 + ks[1]
    x0, x1 = rounds(x0, x1, _ROT_A)
    x0 = x0 + ks[1]; x1 = x1 + np.uint32(ks[2] + np.uint32(1))
    x0, x1 = rounds(x0, x1, _ROT_B)
    x0 = x0 + ks[2]; x1 = x1 + np.uint32(ks[0] + np.uint32(2))
    x0, x1 = rounds(x0, x1, _ROT_A)
    x0 = x0 + ks[0]; x1 = x1 + np.uint32(ks[1] + np.uint32(3))
    x0, x1 = rounds(x0, x1, _ROT_B)
    x0 = x0 + ks[1]; x1 = x1 + np.uint32(ks[2] + np.uint32(4))
    x0, x1 = rounds(x0, x1, _ROT_A)
    x0 = x0 + ks[2]; x1 = x1 + np.uint32(ks[0] + np.uint32(5))
    return x0 ^ x1


def _rvq_kernel(x_ref, cw_ref, cn_ref, quant_ref, probs_ref, idx_ref, loss_ref):
    t = pl.program_id(0)
    fr = x_ref[...]
    cw = cw_ref[...]
    cn = cn_ref[...]

    ci = jax.lax.broadcasted_iota(jnp.int32, (R, NE), 1)
    li = jax.lax.broadcasted_iota(jnp.int32, (R, NE), 0)
    base = t * (R * NE)
    p_u32 = (base + li * NE + ci).astype(jnp.uint32)

    quant = jnp.zeros((R, D), jnp.float32)
    loss_acc = jnp.float32(0.0)

    for s in range(NQ):
        bits = _jnp_threefry_bits(_KEYS[s][0], _KEYS[s][1], p_u32)
        fb = (bits >> np.uint32(9)) | np.uint32(0x3F800000)
        f = jax.lax.bitcast_convert_type(fb, jnp.float32) - jnp.float32(1.0)
        u = jnp.maximum(_TINY, f * (jnp.float32(1.0) - _TINY) + _TINY)
        g = -jnp.log(-jnp.log(u))

        rn = jnp.sum(fr * fr, axis=1, keepdims=True)
        m = jax.lax.dot_general(fr, cw, (((1,), (1,)), ((), ())))
        d = (rn + cn) - 2.0 * m
        logits = -d

        lmax = jnp.max(logits, axis=1, keepdims=True)
        e = jnp.exp(logits - lmax)
        probs = e / jnp.sum(e, axis=1, keepdims=True)
        probs_ref[s, :, :] = probs

        v = g + logits
        vmax = jnp.max(v, axis=1, keepdims=True)
        idx = jnp.min(jnp.where(v == vmax, ci, NE), axis=1, keepdims=True)
        idx_ref[s] = idx

        onehot = (ci == idx).astype(jnp.float32)
        qr = jax.lax.dot_general(onehot, cw, (((1,), (0,)), ((), ())))

        quant = quant + qr
        diff = qr - fr
        loss_acc = loss_acc + jnp.sum(diff * diff)
        fr = fr - qr

    quant_ref[...] = quant
    loss_ref[...] = loss_acc.reshape(1, 1, 1)


def kernel(x, codebook_weight):
    xt = jnp.transpose(x, (0, 2, 3, 1)).reshape(ROWS, D)
    cn = jnp.sum(codebook_weight ** 2, axis=1).reshape(1, NE)

    quant, probs, idx, loss = pl.pallas_call(
        _rvq_kernel,
        grid=(TILES,),
        in_specs=[
            pl.BlockSpec((R, D), lambda t: (t, 0)),
            pl.BlockSpec((NE, D), lambda t: (0, 0)),
            pl.BlockSpec((1, NE), lambda t: (0, 0)),
        ],
        out_specs=[
            pl.BlockSpec((R, D), lambda t: (t, 0)),
            pl.BlockSpec((NQ, R, NE), lambda t: (0, t, 0)),
            pl.BlockSpec((NQ, R, 1), lambda t: (0, t, 0)),
            pl.BlockSpec((1, 1, 1), lambda t: (t, 0, 0)),
        ],
        out_shape=[
            jax.ShapeDtypeStruct((ROWS, D), jnp.float32),
            jax.ShapeDtypeStruct((NQ, ROWS, NE), jnp.float32),
            jax.ShapeDtypeStruct((NQ, ROWS, 1), jnp.int32),
            jax.ShapeDtypeStruct((TILES, 1, 1), jnp.float32),
        ],
        compiler_params=pltpu.CompilerParams(
            dimension_semantics=("parallel",)),
    )(xt, codebook_weight, cn)

    B, C, H, W = x.shape
    quantized = jnp.transpose(quant.reshape(B, H, W, C), (0, 3, 1, 2))
    total_loss = (jnp.float32(1.25) * jnp.sum(loss) / jnp.float32(x.size))
    indices = tuple(idx[s, :, 0].reshape(B, H, W) for s in range(NQ))
    soft_targets = tuple(probs[s].reshape(B, H, W, NE) for s in range(NQ))
    return (quantized, total_loss, indices, soft_targets)


# speedup vs baseline: 1.0977x; 1.0967x over previous
import numpy as np
import jax
import jax.numpy as jnp
from jax.experimental import pallas as pl
from jax.experimental.pallas import tpu as pltpu

NE = 8192
D = 64
NQ = 4
ROWS = 4 * 24 * 24
R = 64
TILES = ROWS // R

_ROT_A = (13, 15, 26, 6)
_ROT_B = (17, 29, 16, 24)


def _np_threefry2x32(k1, k2, x0, x1):
    k1 = np.uint32(k1); k2 = np.uint32(k2)
    ks = (k1, k2, np.uint32(k1 ^ k2 ^ np.uint32(0x1BD11BDA)))
    x0 = np.uint32(x0); x1 = np.uint32(x1)

    def rl(x, r):
        return np.uint32((x << np.uint32(r)) | (x >> np.uint32(32 - r)))

    def rounds(x0, x1, rs):
        for r in rs:
            x0 = np.uint32(x0 + x1)
            x1 = np.uint32(x0 ^ rl(x1, r))
        return x0, x1

    x0 = np.uint32(x0 + ks[0]); x1 = np.uint32(x1 + ks[1])
    x0, x1 = rounds(x0, x1, _ROT_A)
    x0 = np.uint32(x0 + ks[1]); x1 = np.uint32(x1 + ks[2] + np.uint32(1))
    x0, x1 = rounds(x0, x1, _ROT_B)
    x0 = np.uint32(x0 + ks[2]); x1 = np.uint32(x1 + ks[0] + np.uint32(2))
    x0, x1 = rounds(x0, x1, _ROT_A)
    x0 = np.uint32(x0 + ks[0]); x1 = np.uint32(x1 + ks[1] + np.uint32(3))
    x0, x1 = rounds(x0, x1, _ROT_B)
    x0 = np.uint32(x0 + ks[1]); x1 = np.uint32(x1 + ks[2] + np.uint32(4))
    x0, x1 = rounds(x0, x1, _ROT_A)
    x0 = np.uint32(x0 + ks[2]); x1 = np.uint32(x1 + ks[0] + np.uint32(5))
    return x0, x1


def _step_keys():
    np.seterr(over="ignore")
    base = (np.uint32(0), np.uint32(1234))
    keys = []
    for i in range(NQ):
        o0, o1 = _np_threefry2x32(base[0], base[1], np.uint32(0), np.uint32(i))
        keys.append((int(o0), int(o1)))
    return keys


_KEYS = _step_keys()
_TINY = np.float32(np.finfo(np.float32).tiny)


def _jnp_threefry_bits(k1, k2, x1):
    ks = (np.uint32(k1), np.uint32(k2),
          np.uint32(np.uint32(k1) ^ np.uint32(k2) ^ np.uint32(0x1BD11BDA)))

    def rl(x, r):
        return (x << np.uint32(r)) | (x >> np.uint32(32 - r))

    def rounds(x0, x1, rs):
        for r in rs:
            x0 = x0 + x1
            x1 = x0 ^ rl(x1, r)
        return x0, x1

    x0 = jnp.full_like(x1, ks[0])
    x1 = x1 + ks[1]
    x0, x1 = rounds(x0, x1, _ROT_A)
    x0 = x0 + ks[1]; x1 = x1 + np.uint32(ks[2] + np.uint32(1))
    x0, x1 = rounds(x0, x1, _ROT_B)
    x0 = x0 + ks[2]; x1 = x1 + np.uint32(ks[0] + np.uint32(2))
    x0, x1 = rounds(x0, x1, _ROT_A)
    x0 = x0 + ks[0]; x1 = x1 + np.uint32(ks[1] + np.uint32(3))
    x0, x1 = rounds(x0, x1, _ROT_B)
    x0 = x0 + ks[1]; x1 = x1 + np.uint32(ks[2] + np.uint32(4))
    x0, x1 = rounds(x0, x1, _ROT_A)
    x0 = x0 + ks[2]; x1 = x1 + np.uint32(ks[0] + np.uint32(5))
    return x0 ^ x1


def _rvq_kernel(x_ref, cw_ref, cn_ref, quant_ref,
                p0_ref, p1_ref, p2_ref, p3_ref,
                i0_ref, i1_ref, i2_ref, i3_ref, loss_ref):
    probs_refs = (p0_ref, p1_ref, p2_ref, p3_ref)
    idx_refs = (i0_ref, i1_ref, i2_ref, i3_ref)
    t = pl.program_id(0)
    fr = x_ref[...]
    cw = cw_ref[...]
    cn = cn_ref[...]

    ci = jax.lax.broadcasted_iota(jnp.int32, (R, NE), 1)
    li = jax.lax.broadcasted_iota(jnp.int32, (R, NE), 0)
    base = t * (R * NE)
    p_u32 = (base + li * NE + ci).astype(jnp.uint32)

    quant = jnp.zeros((R, D), jnp.float32)
    loss_acc = jnp.float32(0.0)

    for s in range(NQ):
        bits = _jnp_threefry_bits(_KEYS[s][0], _KEYS[s][1], p_u32)
        fb = (bits >> np.uint32(9)) | np.uint32(0x3F800000)
        f = jax.lax.bitcast_convert_type(fb, jnp.float32) - jnp.float32(1.0)
        u = jnp.maximum(_TINY, f * (jnp.float32(1.0) - _TINY) + _TINY)
        g = -jnp.log(-jnp.log(u))

        rn = jnp.sum(fr * fr, axis=1, keepdims=True)
        m = jax.lax.dot_general(fr, cw, (((1,), (1,)), ((), ())))
        d = (rn + cn) - 2.0 * m
        logits = -d

        lmax = jnp.max(logits, axis=1, keepdims=True)
        e = jnp.exp(logits - lmax)
        probs = e / jnp.sum(e, axis=1, keepdims=True)
        probs_refs[s][...] = probs

        v = g + logits
        vmax = jnp.max(v, axis=1, keepdims=True)
        idx = jnp.min(jnp.where(v == vmax, ci, NE), axis=1, keepdims=True)
        idx_refs[s][...] = idx

        onehot = (ci == idx).astype(jnp.float32)
        qr = jax.lax.dot_general(onehot, cw, (((1,), (0,)), ((), ())))

        quant = quant + qr
        diff = qr - fr
        loss_acc = loss_acc + jnp.sum(diff * diff)
        fr = fr - qr

    quant_ref[...] = quant
    loss_ref[...] = loss_acc.reshape(1, 1, 1)


def kernel(x, codebook_weight):
    xt = jnp.transpose(x, (0, 2, 3, 1)).reshape(ROWS, D)
    cn = jnp.sum(codebook_weight ** 2, axis=1).reshape(1, NE)

    outs = pl.pallas_call(
        _rvq_kernel,
        grid=(TILES,),
        in_specs=[
            pl.BlockSpec((R, D), lambda t: (t, 0)),
            pl.BlockSpec((NE, D), lambda t: (0, 0)),
            pl.BlockSpec((1, NE), lambda t: (0, 0)),
        ],
        out_specs=(
            [pl.BlockSpec((R, D), lambda t: (t, 0))]
            + [pl.BlockSpec((R, NE), lambda t: (t, 0)) for _ in range(NQ)]
            + [pl.BlockSpec((R, 1), lambda t: (t, 0)) for _ in range(NQ)]
            + [pl.BlockSpec((1, 1, 1), lambda t: (t, 0, 0))]
        ),
        out_shape=(
            [jax.ShapeDtypeStruct((ROWS, D), jnp.float32)]
            + [jax.ShapeDtypeStruct((ROWS, NE), jnp.float32) for _ in range(NQ)]
            + [jax.ShapeDtypeStruct((ROWS, 1), jnp.int32) for _ in range(NQ)]
            + [jax.ShapeDtypeStruct((TILES, 1, 1), jnp.float32)]
        ),
        compiler_params=pltpu.CompilerParams(
            dimension_semantics=("parallel",)),
    )(xt, codebook_weight, cn)

    quant = outs[0]
    probs_list = outs[1:1 + NQ]
    idx_list = outs[1 + NQ:1 + 2 * NQ]
    loss = outs[1 + 2 * NQ]

    B, C, H, W = x.shape
    quantized = jnp.transpose(quant.reshape(B, H, W, C), (0, 3, 1, 2))
    total_loss = (jnp.float32(1.25) * jnp.sum(loss) / jnp.float32(x.size))
    indices = tuple(i.reshape(B, H, W) for i in idx_list)
    soft_targets = tuple(p.reshape(B, H, W, NE) for p in probs_list)
    return (quantized, total_loss, indices, soft_targets)


# speedup vs baseline: 1.1912x; 1.0852x over previous
import numpy as np
import jax
import jax.numpy as jnp
from jax.experimental import pallas as pl
from jax.experimental.pallas import tpu as pltpu

NE = 8192
D = 64
NQ = 4
ROWS = 4 * 24 * 24
R = 96
TILES = ROWS // R

_ROT_A = (13, 15, 26, 6)
_ROT_B = (17, 29, 16, 24)


def _np_threefry2x32(k1, k2, x0, x1):
    k1 = np.uint32(k1); k2 = np.uint32(k2)
    ks = (k1, k2, np.uint32(k1 ^ k2 ^ np.uint32(0x1BD11BDA)))
    x0 = np.uint32(x0); x1 = np.uint32(x1)

    def rl(x, r):
        return np.uint32((x << np.uint32(r)) | (x >> np.uint32(32 - r)))

    def rounds(x0, x1, rs):
        for r in rs:
            x0 = np.uint32(x0 + x1)
            x1 = np.uint32(x0 ^ rl(x1, r))
        return x0, x1

    x0 = np.uint32(x0 + ks[0]); x1 = np.uint32(x1 + ks[1])
    x0, x1 = rounds(x0, x1, _ROT_A)
    x0 = np.uint32(x0 + ks[1]); x1 = np.uint32(x1 + ks[2] + np.uint32(1))
    x0, x1 = rounds(x0, x1, _ROT_B)
    x0 = np.uint32(x0 + ks[2]); x1 = np.uint32(x1 + ks[0] + np.uint32(2))
    x0, x1 = rounds(x0, x1, _ROT_A)
    x0 = np.uint32(x0 + ks[0]); x1 = np.uint32(x1 + ks[1] + np.uint32(3))
    x0, x1 = rounds(x0, x1, _ROT_B)
    x0 = np.uint32(x0 + ks[1]); x1 = np.uint32(x1 + ks[2] + np.uint32(4))
    x0, x1 = rounds(x0, x1, _ROT_A)
    x0 = np.uint32(x0 + ks[2]); x1 = np.uint32(x1 + ks[0] + np.uint32(5))
    return x0, x1


def _step_keys():
    np.seterr(over="ignore")
    base = (np.uint32(0), np.uint32(1234))
    keys = []
    for i in range(NQ):
        o0, o1 = _np_threefry2x32(base[0], base[1], np.uint32(0), np.uint32(i))
        keys.append((int(o0), int(o1)))
    return keys


_KEYS = _step_keys()
_TINY = np.float32(np.finfo(np.float32).tiny)


def _jnp_threefry_bits(k1, k2, x1):
    ks = (np.uint32(k1), np.uint32(k2),
          np.uint32(np.uint32(k1) ^ np.uint32(k2) ^ np.uint32(0x1BD11BDA)))

    def rl(x, r):
        return (x << np.uint32(r)) | (x >> np.uint32(32 - r))

    def rounds(x0, x1, rs):
        for r in rs:
            x0 = x0 + x1
            x1 = x0 ^ rl(x1, r)
        return x0, x1

    x0 = jnp.full_like(x1, ks[0])
    x1 = x1 + ks[1]
    x0, x1 = rounds(x0, x1, _ROT_A)
    x0 = x0 + ks[1]; x1 = x1 + np.uint32(ks[2] + np.uint32(1))
    x0, x1 = rounds(x0, x1, _ROT_B)
    x0 = x0 + ks[2]; x1 = x1 + np.uint32(ks[0] + np.uint32(2))
    x0, x1 = rounds(x0, x1, _ROT_A)
    x0 = x0 + ks[0]; x1 = x1 + np.uint32(ks[1] + np.uint32(3))
    x0, x1 = rounds(x0, x1, _ROT_B)
    x0 = x0 + ks[1]; x1 = x1 + np.uint32(ks[2] + np.uint32(4))
    x0, x1 = rounds(x0, x1, _ROT_A)
    x0 = x0 + ks[2]; x1 = x1 + np.uint32(ks[0] + np.uint32(5))
    return x0 ^ x1


def _rvq_kernel(x_ref, cw_ref, cn_ref, quant_ref,
                p0_ref, p1_ref, p2_ref, p3_ref,
                i0_ref, i1_ref, i2_ref, i3_ref, loss_ref):
    probs_refs = (p0_ref, p1_ref, p2_ref, p3_ref)
    idx_refs = (i0_ref, i1_ref, i2_ref, i3_ref)
    t = pl.program_id(0)
    fr = x_ref[...]
    cw = cw_ref[...]
    cn = cn_ref[...]

    ci = jax.lax.broadcasted_iota(jnp.int32, (R, NE), 1)
    li = jax.lax.broadcasted_iota(jnp.int32, (R, NE), 0)
    base = t * (R * NE)
    p_u32 = (base + li * NE + ci).astype(jnp.uint32)

    quant = jnp.zeros((R, D), jnp.float32)
    loss_acc = jnp.float32(0.0)

    for s in range(NQ):
        bits = _jnp_threefry_bits(_KEYS[s][0], _KEYS[s][1], p_u32)
        fb = (bits >> np.uint32(9)) | np.uint32(0x3F800000)
        f = jax.lax.bitcast_convert_type(fb, jnp.float32) - jnp.float32(1.0)
        u = jnp.maximum(_TINY, f * (jnp.float32(1.0) - _TINY) + _TINY)
        g = -jnp.log(-jnp.log(u))

        rn = jnp.sum(fr * fr, axis=1, keepdims=True)
        m = jax.lax.dot_general(fr, cw, (((1,), (1,)), ((), ())))
        d = (rn + cn) - 2.0 * m
        logits = -d

        lmax = jnp.max(logits, axis=1, keepdims=True)
        e = jnp.exp(logits - lmax)
        probs = e / jnp.sum(e, axis=1, keepdims=True)
        probs_refs[s][...] = probs

        v = g + logits
        vmax = jnp.max(v, axis=1, keepdims=True)
        idx = jnp.min(jnp.where(v == vmax, ci, NE), axis=1, keepdims=True)
        idx_refs[s][...] = idx

        onehot = (ci == idx).astype(jnp.float32)
        qr = jax.lax.dot_general(onehot, cw, (((1,), (0,)), ((), ())))

        quant = quant + qr
        diff = qr - fr
        loss_acc = loss_acc + jnp.sum(diff * diff)
        fr = fr - qr

    quant_ref[...] = quant
    loss_ref[...] = loss_acc.reshape(1, 1, 1)


def kernel(x, codebook_weight):
    xt = jnp.transpose(x, (0, 2, 3, 1)).reshape(ROWS, D)
    cn = jnp.sum(codebook_weight ** 2, axis=1).reshape(1, NE)

    outs = pl.pallas_call(
        _rvq_kernel,
        grid=(TILES,),
        in_specs=[
            pl.BlockSpec((R, D), lambda t: (t, 0)),
            pl.BlockSpec((NE, D), lambda t: (0, 0)),
            pl.BlockSpec((1, NE), lambda t: (0, 0)),
        ],
        out_specs=(
            [pl.BlockSpec((R, D), lambda t: (t, 0))]
            + [pl.BlockSpec((R, NE), lambda t: (t, 0)) for _ in range(NQ)]
            + [pl.BlockSpec((R, 1), lambda t: (t, 0)) for _ in range(NQ)]
            + [pl.BlockSpec((1, 1, 1), lambda t: (t, 0, 0))]
        ),
        out_shape=(
            [jax.ShapeDtypeStruct((ROWS, D), jnp.float32)]
            + [jax.ShapeDtypeStruct((ROWS, NE), jnp.float32) for _ in range(NQ)]
            + [jax.ShapeDtypeStruct((ROWS, 1), jnp.int32) for _ in range(NQ)]
            + [jax.ShapeDtypeStruct((TILES, 1, 1), jnp.float32)]
        ),
        compiler_params=pltpu.CompilerParams(
            dimension_semantics=("parallel",)),
    )(xt, codebook_weight, cn)

    quant = outs[0]
    probs_list = outs[1:1 + NQ]
    idx_list = outs[1 + NQ:1 + 2 * NQ]
    loss = outs[1 + 2 * NQ]

    B, C, H, W = x.shape
    quantized = jnp.transpose(quant.reshape(B, H, W, C), (0, 3, 1, 2))
    total_loss = (jnp.float32(1.25) * jnp.sum(loss) / jnp.float32(x.size))
    indices = tuple(i.reshape(B, H, W) for i in idx_list)
    soft_targets = tuple(p.reshape(B, H, W, NE) for p in probs_list)
    return (quantized, total_loss, indices, soft_targets)


# speedup vs baseline: 1.1964x; 1.0043x over previous
import numpy as np
import jax
import jax.numpy as jnp
from jax.experimental import pallas as pl
from jax.experimental.pallas import tpu as pltpu

NE = 8192
D = 64
NQ = 4
ROWS = 4 * 24 * 24
R = 96
TILES = ROWS // R

_ROT_A = (13, 15, 26, 6)
_ROT_B = (17, 29, 16, 24)


def _np_threefry2x32(k1, k2, x0, x1):
    k1 = np.uint32(k1); k2 = np.uint32(k2)
    ks = (k1, k2, np.uint32(k1 ^ k2 ^ np.uint32(0x1BD11BDA)))
    x0 = np.uint32(x0); x1 = np.uint32(x1)

    def rl(x, r):
        return np.uint32((x << np.uint32(r)) | (x >> np.uint32(32 - r)))

    def rounds(x0, x1, rs):
        for r in rs:
            x0 = np.uint32(x0 + x1)
            x1 = np.uint32(x0 ^ rl(x1, r))
        return x0, x1

    x0 = np.uint32(x0 + ks[0]); x1 = np.uint32(x1 + ks[1])
    x0, x1 = rounds(x0, x1, _ROT_A)
    x0 = np.uint32(x0 + ks[1]); x1 = np.uint32(x1 + ks[2] + np.uint32(1))
    x0, x1 = rounds(x0, x1, _ROT_B)
    x0 = np.uint32(x0 + ks[2]); x1 = np.uint32(x1 + ks[0] + np.uint32(2))
    x0, x1 = rounds(x0, x1, _ROT_A)
    x0 = np.uint32(x0 + ks[0]); x1 = np.uint32(x1 + ks[1] + np.uint32(3))
    x0, x1 = rounds(x0, x1, _ROT_B)
    x0 = np.uint32(x0 + ks[1]); x1 = np.uint32(x1 + ks[2] + np.uint32(4))
    x0, x1 = rounds(x0, x1, _ROT_A)
    x0 = np.uint32(x0 + ks[2]); x1 = np.uint32(x1 + ks[0] + np.uint32(5))
    return x0, x1


def _step_keys():
    np.seterr(over="ignore")
    base = (np.uint32(0), np.uint32(1234))
    keys = []
    for i in range(NQ):
        o0, o1 = _np_threefry2x32(base[0], base[1], np.uint32(0), np.uint32(i))
        keys.append((int(o0), int(o1)))
    return keys


_KEYS = _step_keys()
_TINY = np.float32(np.finfo(np.float32).tiny)


def _jnp_threefry_bits(k1, k2, x1):
    ks = (np.uint32(k1), np.uint32(k2),
          np.uint32(np.uint32(k1) ^ np.uint32(k2) ^ np.uint32(0x1BD11BDA)))

    def rl(x, r):
        return (x << np.uint32(r)) | (x >> np.uint32(32 - r))

    def rounds(x0, x1, rs):
        for r in rs:
            x0 = x0 + x1
            x1 = x0 ^ rl(x1, r)
        return x0, x1

    x0 = jnp.full_like(x1, ks[0])
    x1 = x1 + ks[1]
    x0, x1 = rounds(x0, x1, _ROT_A)
    x0 = x0 + ks[1]; x1 = x1 + np.uint32(ks[2] + np.uint32(1))
    x0, x1 = rounds(x0, x1, _ROT_B)
    x0 = x0 + ks[2]; x1 = x1 + np.uint32(ks[0] + np.uint32(2))
    x0, x1 = rounds(x0, x1, _ROT_A)
    x0 = x0 + ks[0]; x1 = x1 + np.uint32(ks[1] + np.uint32(3))
    x0, x1 = rounds(x0, x1, _ROT_B)
    x0 = x0 + ks[1]; x1 = x1 + np.uint32(ks[2] + np.uint32(4))
    x0, x1 = rounds(x0, x1, _ROT_A)
    x0 = x0 + ks[2]; x1 = x1 + np.uint32(ks[0] + np.uint32(5))
    return x0 ^ x1


def _rvq_kernel(x_ref, cw_ref, cn_ref, quant_ref,
                p0_ref, p1_ref, p2_ref, p3_ref,
                i0_ref, i1_ref, i2_ref, i3_ref, loss_ref):
    probs_refs = (p0_ref, p1_ref, p2_ref, p3_ref)
    idx_refs = (i0_ref, i1_ref, i2_ref, i3_ref)
    t = pl.program_id(0)
    fr = x_ref[...]
    cw = cw_ref[...]
    cn = cn_ref[...]

    ci = jax.lax.broadcasted_iota(jnp.int32, (R, NE), 1)
    li = jax.lax.broadcasted_iota(jnp.int32, (R, NE), 0)
    base = t * (R * NE)
    p_u32 = (base + li * NE + ci).astype(jnp.uint32)

    quant = jnp.zeros((R, D), jnp.float32)
    loss_acc = jnp.float32(0.0)

    for s in range(NQ):
        bits = _jnp_threefry_bits(_KEYS[s][0], _KEYS[s][1], p_u32)
        fb = (bits >> np.uint32(9)) | np.uint32(0x3F800000)
        f = jax.lax.bitcast_convert_type(fb, jnp.float32) - jnp.float32(1.0)
        u = jnp.maximum(f, _TINY)
        lt = jnp.log(-jnp.log(u))

        rn = jnp.sum(fr * fr, axis=1, keepdims=True)
        m = jax.lax.dot_general(fr, cw, (((1,), (1,)), ((), ())))
        d = (rn + cn) - 2.0 * m

        dmin = jnp.min(d, axis=1, keepdims=True)
        e = jnp.exp(dmin - d)
        probs = e / jnp.sum(e, axis=1, keepdims=True)
        probs_refs[s][...] = probs

        v = -(lt + d)
        vmax = jnp.max(v, axis=1, keepdims=True)
        idx = jnp.min(jnp.where(v == vmax, ci, NE), axis=1, keepdims=True)
        idx_refs[s][...] = idx

        onehot = (ci == idx).astype(jnp.float32)
        qr = jax.lax.dot_general(onehot, cw, (((1,), (0,)), ((), ())))

        quant = quant + qr
        diff = qr - fr
        loss_acc = loss_acc + jnp.sum(diff * diff)
        fr = fr - qr

    quant_ref[...] = quant
    loss_ref[...] = loss_acc.reshape(1, 1, 1)


def kernel(x, codebook_weight):
    xt = jnp.transpose(x, (0, 2, 3, 1)).reshape(ROWS, D)
    cn = jnp.sum(codebook_weight ** 2, axis=1).reshape(1, NE)

    outs = pl.pallas_call(
        _rvq_kernel,
        grid=(TILES,),
        in_specs=[
            pl.BlockSpec((R, D), lambda t: (t, 0)),
            pl.BlockSpec((NE, D), lambda t: (0, 0)),
            pl.BlockSpec((1, NE), lambda t: (0, 0)),
        ],
        out_specs=(
            [pl.BlockSpec((R, D), lambda t: (t, 0))]
            + [pl.BlockSpec((R, NE), lambda t: (t, 0)) for _ in range(NQ)]
            + [pl.BlockSpec((R, 1), lambda t: (t, 0)) for _ in range(NQ)]
            + [pl.BlockSpec((1, 1, 1), lambda t: (t, 0, 0))]
        ),
        out_shape=(
            [jax.ShapeDtypeStruct((ROWS, D), jnp.float32)]
            + [jax.ShapeDtypeStruct((ROWS, NE), jnp.float32) for _ in range(NQ)]
            + [jax.ShapeDtypeStruct((ROWS, 1), jnp.int32) for _ in range(NQ)]
            + [jax.ShapeDtypeStruct((TILES, 1, 1), jnp.float32)]
        ),
        compiler_params=pltpu.CompilerParams(
            dimension_semantics=("parallel",)),
    )(xt, codebook_weight, cn)

    quant = outs[0]
    probs_list = outs[1:1 + NQ]
    idx_list = outs[1 + NQ:1 + 2 * NQ]
    loss = outs[1 + 2 * NQ]

    B, C, H, W = x.shape
    quantized = jnp.transpose(quant.reshape(B, H, W, C), (0, 3, 1, 2))
    total_loss = (jnp.float32(1.25) * jnp.sum(loss) / jnp.float32(x.size))
    indices = tuple(i.reshape(B, H, W) for i in idx_list)
    soft_targets = tuple(p.reshape(B, H, W, NE) for p in probs_list)
    return (quantized, total_loss, indices, soft_targets)


# speedup vs baseline: 1.3824x; 1.1555x over previous
import numpy as np
import jax
import jax.numpy as jnp
from jax.experimental import pallas as pl
from jax.experimental.pallas import tpu as pltpu

NE = 8192
D = 64
NQ = 4
ROWS = 4 * 24 * 24
R = 96
TILES = ROWS // R

_ROT_A = (13, 15, 26, 6)
_ROT_B = (17, 29, 16, 24)


def _np_threefry2x32(k1, k2, x0, x1):
    k1 = np.uint32(k1); k2 = np.uint32(k2)
    ks = (k1, k2, np.uint32(k1 ^ k2 ^ np.uint32(0x1BD11BDA)))
    x0 = np.uint32(x0); x1 = np.uint32(x1)

    def rl(x, r):
        return np.uint32((x << np.uint32(r)) | (x >> np.uint32(32 - r)))

    def rounds(x0, x1, rs):
        for r in rs:
            x0 = np.uint32(x0 + x1)
            x1 = np.uint32(x0 ^ rl(x1, r))
        return x0, x1

    x0 = np.uint32(x0 + ks[0]); x1 = np.uint32(x1 + ks[1])
    x0, x1 = rounds(x0, x1, _ROT_A)
    x0 = np.uint32(x0 + ks[1]); x1 = np.uint32(x1 + ks[2] + np.uint32(1))
    x0, x1 = rounds(x0, x1, _ROT_B)
    x0 = np.uint32(x0 + ks[2]); x1 = np.uint32(x1 + ks[0] + np.uint32(2))
    x0, x1 = rounds(x0, x1, _ROT_A)
    x0 = np.uint32(x0 + ks[0]); x1 = np.uint32(x1 + ks[1] + np.uint32(3))
    x0, x1 = rounds(x0, x1, _ROT_B)
    x0 = np.uint32(x0 + ks[1]); x1 = np.uint32(x1 + ks[2] + np.uint32(4))
    x0, x1 = rounds(x0, x1, _ROT_A)
    x0 = np.uint32(x0 + ks[2]); x1 = np.uint32(x1 + ks[0] + np.uint32(5))
    return x0, x1


def _step_keys():
    np.seterr(over="ignore")
    keys = []
    for i in range(NQ):
        o0, o1 = _np_threefry2x32(np.uint32(0), np.uint32(1234),
                                  np.uint32(0), np.uint32(i))
        keys.append((int(o0), int(o1)))
    return keys


_KEYS = _step_keys()
_TINY = np.float32(np.finfo(np.float32).tiny)


def _jnp_threefry_bits(k1, k2, x1):
    kx = k1 ^ k2 ^ np.uint32(0x1BD11BDA)
    ks = (k1, k2, kx)

    def rl(x, r):
        return (x << np.uint32(r)) | (x >> np.uint32(32 - r))

    def rounds(x0, x1, rs):
        for r in rs:
            x0 = x0 + x1
            x1 = x0 ^ rl(x1, r)
        return x0, x1

    x0 = jnp.broadcast_to(ks[0], x1.shape)
    x1 = x1 + ks[1]
    x0, x1 = rounds(x0, x1, _ROT_A)
    x0 = x0 + ks[1]; x1 = x1 + (ks[2] + np.uint32(1))
    x0, x1 = rounds(x0, x1, _ROT_B)
    x0 = x0 + ks[2]; x1 = x1 + (ks[0] + np.uint32(2))
    x0, x1 = rounds(x0, x1, _ROT_A)
    x0 = x0 + ks[0]; x1 = x1 + (ks[1] + np.uint32(3))
    x0, x1 = rounds(x0, x1, _ROT_B)
    x0 = x0 + ks[1]; x1 = x1 + (ks[2] + np.uint32(4))
    x0, x1 = rounds(x0, x1, _ROT_A)
    x0 = x0 + ks[2]; x1 = x1 + (ks[0] + np.uint32(5))
    return x0 ^ x1


def _rvq_kernel(keys_ref, x_ref, cw_ref, cn_ref, quant_ref,
                p0_ref, p1_ref, p2_ref, p3_ref,
                i0_ref, i1_ref, i2_ref, i3_ref, loss_ref,
                fr_s, qacc_s):
    probs_refs = (p0_ref, p1_ref, p2_ref, p3_ref)
    idx_refs = (i0_ref, i1_ref, i2_ref, i3_ref)
    t = pl.program_id(0)
    s = pl.program_id(1)

    @pl.when(s == 0)
    def _():
        fr_s[...] = x_ref[...]
        qacc_s[...] = jnp.zeros((R, D), jnp.float32)

    fr = fr_s[...]
    cw = cw_ref[...]
    cn = cn_ref[...]
    k1 = keys_ref[s, 0]
    k2 = keys_ref[s, 1]

    ci = jax.lax.broadcasted_iota(jnp.int32, (R, NE), 1)
    li = jax.lax.broadcasted_iota(jnp.int32, (R, 1), 0)
    p_u32 = ((t * (R * NE) + li * NE) + ci).astype(jnp.uint32)

    bits = _jnp_threefry_bits(k1, k2, p_u32)
    fb = (bits >> np.uint32(9)) | np.uint32(0x3F800000)
    f = jax.lax.bitcast_convert_type(fb, jnp.float32) - jnp.float32(1.0)
    u = jnp.maximum(f, _TINY)
    lt = jnp.log(-jnp.log(u))

    rn = jnp.sum(fr * fr, axis=1, keepdims=True)
    m = jax.lax.dot_general(fr, cw, (((1,), (1,)), ((), ())))
    d = (rn + cn) - 2.0 * m

    dmin = jnp.min(d, axis=1, keepdims=True)
    e = jnp.exp(dmin - d)
    probs = e / jnp.sum(e, axis=1, keepdims=True)

    v = -(lt + d)
    vmax = jnp.max(v, axis=1, keepdims=True)
    idx = jnp.min(jnp.where(v == vmax, ci, NE), axis=1, keepdims=True)

    for j in range(NQ):
        @pl.when(s == j)
        def _(j=j):
            probs_refs[j][...] = probs
            idx_refs[j][...] = idx

    onehot = (ci == idx).astype(jnp.float32)
    qr = jax.lax.dot_general(onehot, cw, (((1,), (0,)), ((), ())))

    qacc_s[...] += qr
    diff = qr - fr
    ls = jnp.sum(diff * diff).reshape(1, 1, 1)
    fr_s[...] = fr - qr

    @pl.when(s == 0)
    def _():
        loss_ref[...] = ls

    @pl.when(s != 0)
    def _():
        loss_ref[...] += ls

    @pl.when(s == NQ - 1)
    def _():
        quant_ref[...] = qacc_s[...]


def kernel(x, codebook_weight):
    xt = jnp.transpose(x, (0, 2, 3, 1)).reshape(ROWS, D)
    cn = jnp.sum(codebook_weight ** 2, axis=1).reshape(1, NE)
    keys = jnp.asarray(np.array(_KEYS, dtype=np.uint32))

    outs = pl.pallas_call(
        _rvq_kernel,
        grid=(TILES, NQ),
        in_specs=[
            pl.BlockSpec(memory_space=pltpu.SMEM),
            pl.BlockSpec((R, D), lambda t, s: (t, 0)),
            pl.BlockSpec((NE, D), lambda t, s: (0, 0)),
            pl.BlockSpec((1, NE), lambda t, s: (0, 0)),
        ],
        out_specs=(
            [pl.BlockSpec((R, D), lambda t, s: (t, 0))]
            + [pl.BlockSpec((R, NE), lambda t, s: (t, 0)) for _ in range(NQ)]
            + [pl.BlockSpec((R, 1), lambda t, s: (t, 0)) for _ in range(NQ)]
            + [pl.BlockSpec((1, 1, 1), lambda t, s: (t, 0, 0))]
        ),
        out_shape=(
            [jax.ShapeDtypeStruct((ROWS, D), jnp.float32)]
            + [jax.ShapeDtypeStruct((ROWS, NE), jnp.float32) for _ in range(NQ)]
            + [jax.ShapeDtypeStruct((ROWS, 1), jnp.int32) for _ in range(NQ)]
            + [jax.ShapeDtypeStruct((TILES, 1, 1), jnp.float32)]
        ),
        scratch_shapes=[
            pltpu.VMEM((R, D), jnp.float32),
            pltpu.VMEM((R, D), jnp.float32),
        ],
    )(keys, xt, codebook_weight, cn)

    quant = outs[0]
    probs_list = outs[1:1 + NQ]
    idx_list = outs[1 + NQ:1 + 2 * NQ]
    loss = outs[1 + 2 * NQ]

    B, C, H, W = x.shape
    quantized = jnp.transpose(quant.reshape(B, H, W, C), (0, 3, 1, 2))
    total_loss = (jnp.float32(1.25) * jnp.sum(loss) / jnp.float32(x.size))
    indices = tuple(i.reshape(B, H, W) for i in idx_list)
    soft_targets = tuple(p.reshape(B, H, W, NE) for p in probs_list)
    return (quantized, total_loss, indices, soft_targets)


# speedup vs baseline: 1.3839x; 1.0011x over previous
import numpy as np
import jax
import jax.numpy as jnp
from jax.experimental import pallas as pl
from jax.experimental.pallas import tpu as pltpu

NE = 8192
D = 64
NQ = 4
ROWS = 4 * 24 * 24
R = 128
TILES = ROWS // R

_ROT_A = (13, 15, 26, 6)
_ROT_B = (17, 29, 16, 24)


def _np_threefry2x32(k1, k2, x0, x1):
    k1 = np.uint32(k1); k2 = np.uint32(k2)
    ks = (k1, k2, np.uint32(k1 ^ k2 ^ np.uint32(0x1BD11BDA)))
    x0 = np.uint32(x0); x1 = np.uint32(x1)

    def rl(x, r):
        return np.uint32((x << np.uint32(r)) | (x >> np.uint32(32 - r)))

    def rounds(x0, x1, rs):
        for r in rs:
            x0 = np.uint32(x0 + x1)
            x1 = np.uint32(x0 ^ rl(x1, r))
        return x0, x1

    x0 = np.uint32(x0 + ks[0]); x1 = np.uint32(x1 + ks[1])
    x0, x1 = rounds(x0, x1, _ROT_A)
    x0 = np.uint32(x0 + ks[1]); x1 = np.uint32(x1 + ks[2] + np.uint32(1))
    x0, x1 = rounds(x0, x1, _ROT_B)
    x0 = np.uint32(x0 + ks[2]); x1 = np.uint32(x1 + ks[0] + np.uint32(2))
    x0, x1 = rounds(x0, x1, _ROT_A)
    x0 = np.uint32(x0 + ks[0]); x1 = np.uint32(x1 + ks[1] + np.uint32(3))
    x0, x1 = rounds(x0, x1, _ROT_B)
    x0 = np.uint32(x0 + ks[1]); x1 = np.uint32(x1 + ks[2] + np.uint32(4))
    x0, x1 = rounds(x0, x1, _ROT_A)
    x0 = np.uint32(x0 + ks[2]); x1 = np.uint32(x1 + ks[0] + np.uint32(5))
    return x0, x1


def _step_keys():
    np.seterr(over="ignore")
    keys = []
    for i in range(NQ):
        o0, o1 = _np_threefry2x32(np.uint32(0), np.uint32(1234),
                                  np.uint32(0), np.uint32(i))
        keys.append((int(o0), int(o1)))
    return keys


_KEYS = _step_keys()
_TINY = np.float32(np.finfo(np.float32).tiny)


def _jnp_threefry_bits(k1, k2, x1):
    kx = k1 ^ k2 ^ np.uint32(0x1BD11BDA)
    ks = (k1, k2, kx)

    def rl(x, r):
        return (x << np.uint32(r)) | (x >> np.uint32(32 - r))

    def rounds(x0, x1, rs):
        for r in rs:
            x0 = x0 + x1
            x1 = x0 ^ rl(x1, r)
        return x0, x1

    x0 = jnp.broadcast_to(ks[0], x1.shape)
    x1 = x1 + ks[1]
    x0, x1 = rounds(x0, x1, _ROT_A)
    x0 = x0 + ks[1]; x1 = x1 + (ks[2] + np.uint32(1))
    x0, x1 = rounds(x0, x1, _ROT_B)
    x0 = x0 + ks[2]; x1 = x1 + (ks[0] + np.uint32(2))
    x0, x1 = rounds(x0, x1, _ROT_A)
    x0 = x0 + ks[0]; x1 = x1 + (ks[1] + np.uint32(3))
    x0, x1 = rounds(x0, x1, _ROT_B)
    x0 = x0 + ks[1]; x1 = x1 + (ks[2] + np.uint32(4))
    x0, x1 = rounds(x0, x1, _ROT_A)
    x0 = x0 + ks[2]; x1 = x1 + (ks[0] + np.uint32(5))
    return x0 ^ x1


def _rvq_kernel(keys_ref, x_ref, cw_ref, cn_ref, quant_ref,
                p0_ref, p1_ref, p2_ref, p3_ref,
                i0_ref, i1_ref, i2_ref, i3_ref, loss_ref,
                fr_s, qacc_s):
    probs_refs = (p0_ref, p1_ref, p2_ref, p3_ref)
    idx_refs = (i0_ref, i1_ref, i2_ref, i3_ref)
    t = pl.program_id(0)
    s = pl.program_id(1)

    @pl.when(s == 0)
    def _():
        fr_s[...] = x_ref[...]
        qacc_s[...] = jnp.zeros((R, D), jnp.float32)

    fr = fr_s[...]
    cw = cw_ref[...]
    cn = cn_ref[...]
    k1 = keys_ref[s, 0]
    k2 = keys_ref[s, 1]

    ci = jax.lax.broadcasted_iota(jnp.int32, (R, NE), 1)
    li = jax.lax.broadcasted_iota(jnp.int32, (R, 1), 0)
    p_u32 = ((t * (R * NE) + li * NE) + ci).astype(jnp.uint32)

    bits = _jnp_threefry_bits(k1, k2, p_u32)
    fb = (bits >> np.uint32(9)) | np.uint32(0x3F800000)
    f = jax.lax.bitcast_convert_type(fb, jnp.float32) - jnp.float32(1.0)
    u = jnp.maximum(f, _TINY)
    lt = jnp.log(-jnp.log(u))

    rn = jnp.sum(fr * fr, axis=1, keepdims=True)
    m = jax.lax.dot_general(fr, cw, (((1,), (1,)), ((), ())))
    d = (rn + cn) - 2.0 * m

    dmin = jnp.min(d, axis=1, keepdims=True)
    e = jnp.exp(dmin - d)
    probs = e / jnp.sum(e, axis=1, keepdims=True)

    v = -(lt + d)
    vmax = jnp.max(v, axis=1, keepdims=True)
    idx = jnp.min(jnp.where(v == vmax, ci, NE), axis=1, keepdims=True)

    for j in range(NQ):
        @pl.when(s == j)
        def _(j=j):
            probs_refs[j][...] = probs
            idx_refs[j][...] = idx

    onehot = (ci == idx).astype(jnp.float32)
    qr = jax.lax.dot_general(onehot, cw, (((1,), (0,)), ((), ())))

    qacc_s[...] += qr
    diff = qr - fr
    ls = jnp.sum(diff * diff).reshape(1, 1, 1)
    fr_s[...] = fr - qr

    @pl.when(s == 0)
    def _():
        loss_ref[...] = ls

    @pl.when(s != 0)
    def _():
        loss_ref[...] += ls

    @pl.when(s == NQ - 1)
    def _():
        quant_ref[...] = qacc_s[...]


def kernel(x, codebook_weight):
    xt = jnp.transpose(x, (0, 2, 3, 1)).reshape(ROWS, D)
    cn = jnp.sum(codebook_weight ** 2, axis=1).reshape(1, NE)
    keys = jnp.asarray(np.array(_KEYS, dtype=np.uint32))

    outs = pl.pallas_call(
        _rvq_kernel,
        grid=(TILES, NQ),
        in_specs=[
            pl.BlockSpec(memory_space=pltpu.SMEM),
            pl.BlockSpec((R, D), lambda t, s: (t, 0)),
            pl.BlockSpec((NE, D), lambda t, s: (0, 0)),
            pl.BlockSpec((1, NE), lambda t, s: (0, 0)),
        ],
        out_specs=(
            [pl.BlockSpec((R, D), lambda t, s: (t, 0))]
            + [pl.BlockSpec((R, NE), lambda t, s: (t, 0)) for _ in range(NQ)]
            + [pl.BlockSpec((R, 1), lambda t, s: (t, 0)) for _ in range(NQ)]
            + [pl.BlockSpec((1, 1, 1), lambda t, s: (t, 0, 0))]
        ),
        out_shape=(
            [jax.ShapeDtypeStruct((ROWS, D), jnp.float32)]
            + [jax.ShapeDtypeStruct((ROWS, NE), jnp.float32) for _ in range(NQ)]
            + [jax.ShapeDtypeStruct((ROWS, 1), jnp.int32) for _ in range(NQ)]
            + [jax.ShapeDtypeStruct((TILES, 1, 1), jnp.float32)]
        ),
        scratch_shapes=[
            pltpu.VMEM((R, D), jnp.float32),
            pltpu.VMEM((R, D), jnp.float32),
        ],
    )(keys, xt, codebook_weight, cn)

    quant = outs[0]
    probs_list = outs[1:1 + NQ]
    idx_list = outs[1 + NQ:1 + 2 * NQ]
    loss = outs[1 + 2 * NQ]

    B, C, H, W = x.shape
    quantized = jnp.transpose(quant.reshape(B, H, W, C), (0, 3, 1, 2))
    total_loss = (jnp.float32(1.25) * jnp.sum(loss) / jnp.float32(x.size))
    indices = tuple(i.reshape(B, H, W) for i in idx_list)
    soft_targets = tuple(p.reshape(B, H, W, NE) for p in probs_list)
    return (quantized, total_loss, indices, soft_targets)
